# baseline (speedup 1.0000x reference)
import functools

import jax
import jax.numpy as jnp
from jax import lax
from jax.experimental import pallas as pl
from jax.experimental.pallas import tpu as pltpu
from jax.experimental.pallas import tpu_sc as plsc

NUM_NODES = 4096
H = 16
B = 1024
L = 50
LL = L * L
G = (LL + 15) // 16
GP = 160
LLP = GP * 16
LP = 64
NW = 32
BPW = B // NW
VT = NUM_NODES + 1
HH = H // 2
HALF = HH * LL


def _sc_body(seq_hbm, spd_hbm, embt_hbm, out_hbm,
             seq_v, di_v, mo_v, idx_a, idx_b, spd_a, spd_b, embt_v, out_v,
             sem_a, sem_b, sem_o0, sem_o1):
    wid = lax.axis_index("s") * 2 + lax.axis_index("c")
    b0 = wid * BPW
    pltpu.sync_copy(embt_hbm, embt_v)
    pltpu.sync_copy(seq_hbm.at[pl.ds(b0, BPW)], seq_v)
    iota = lax.iota(jnp.int32, 16)

    def qr_body(g, c):
        di, mo = c
        di_v[pl.ds(g * 16, 16)] = di
        mo_v[pl.ds(g * 16, 16)] = mo
        mo2 = mo + 16
        over = mo2 >= L
        return (di + over.astype(jnp.int32), jnp.where(over, mo2 - L, mo2))

    lax.fori_loop(0, GP, qr_body, (iota * 0, iota))

    def build_idx(t, idx_v):
        row = jnp.broadcast_to(t, (16,)).astype(jnp.int32)

        def idx_body(g, c):
            di = di_v[pl.ds(g * 16, 16)]
            mo = mo_v[pl.ds(g * 16, 16)]
            hi = plsc.load_gather(seq_v, [row, di])
            lo = plsc.load_gather(seq_v, [row, mo])
            idx_v[pl.ds(g * 16, 16)] = hi * NUM_NODES + lo
            return c

        lax.fori_loop(0, GP, idx_body, 0)

    def fill_half(spd_v, h0):
        def g_body(g, c):
            sv = spd_v[pl.ds(g * 16, 16)]
            base = g * 16 + iota
            for hh in range(HH):
                h = h0 + hh
                val = plsc.load_gather(embt_v, [sv + (h * VT)])
                plsc.store_scatter(out_v, [base + (h * LL)], val)
            return c

        lax.fori_loop(0, G - 1, g_body, 0)

        g = G - 1
        sv = spd_v[pl.ds(g * 16, 16)]
        base = g * 16 + iota
        tmask = iota < (LL - g * 16)
        for hh in range(HH):
            h = h0 + hh
            val = plsc.load_gather(embt_v, [sv + (h * VT)])
            plsc.store_scatter(out_v, [base + (h * LL)], val, mask=tmask)

    def fire_spd(idx_v, spd_v, sem):
        return pltpu.async_copy(spd_hbm.at[idx_v], spd_v, sem)

    def fire_half(b, half, sem):
        ob = b * (H * LL) + half * HALF
        return pltpu.async_copy(out_v.at[pl.ds(half * HALF, HALF)],
                                out_hbm.at[pl.ds(ob, HALF)], sem)

    build_idx(0, idx_a)
    fire_spd(idx_a, spd_a, sem_a).wait()

    def pair_body(k, carry):
        te = 2 * k
        to = te + 1
        tn = jnp.minimum(to + 1, BPW - 1)

        build_idx(to, idx_b)
        cp_b = fire_spd(idx_b, spd_b, sem_b)
        fill_half(spd_a, 0)
        cp_o0 = fire_half(b0 + te, 0, sem_o0)
        fill_half(spd_a, HH)
        cp_o1 = fire_half(b0 + te, 1, sem_o1)
        cp_b.wait()

        build_idx(tn, idx_a)
        cp_a = fire_spd(idx_a, spd_a, sem_a)
        cp_o0.wait()
        fill_half(spd_b, 0)
        cp_o0b = fire_half(b0 + to, 0, sem_o0)
        cp_o1.wait()
        fill_half(spd_b, HH)
        cp_o1b = fire_half(b0 + to, 1, sem_o1)
        cp_a.wait()
        cp_o0b.wait()
        cp_o1b.wait()
        return carry

    lax.fori_loop(0, BPW // 2, pair_body, 0)


@functools.partial(
    pl.kernel,
    mesh=plsc.VectorSubcoreMesh(core_axis_name="c", subcore_axis_name="s"),
    compiler_params=pltpu.CompilerParams(needs_layout_passes=False),
    out_type=jax.ShapeDtypeStruct((B * H * LL,), jnp.float32),
    scratch_types=[
        pltpu.VMEM((BPW, LP), jnp.int32),
        pltpu.VMEM((LLP,), jnp.int32),
        pltpu.VMEM((LLP,), jnp.int32),
        pltpu.VMEM((LLP,), jnp.int32),
        pltpu.VMEM((LLP,), jnp.int32),
        pltpu.VMEM((LLP,), jnp.int32),
        pltpu.VMEM((LLP,), jnp.int32),
        pltpu.VMEM((H * VT,), jnp.float32),
        pltpu.VMEM((H * LL,), jnp.float32),
        pltpu.SemaphoreType.DMA,
        pltpu.SemaphoreType.DMA,
        pltpu.SemaphoreType.DMA,
        pltpu.SemaphoreType.DMA,
    ],
)
def _sc_kernel(*refs):
    _sc_body(*refs)


@jax.jit
def kernel(user_seq, spd_table, emb):
    seq = user_seq.astype(jnp.int32)
    seq_p = jnp.zeros((B, LP), jnp.int32).at[:, :L].set(seq)
    spd_flat = spd_table.reshape(-1)
    embt = emb.T.reshape(-1)
    out = _sc_kernel(seq_p, spd_flat, embt)
    return out.reshape(B, H, L, L)


# speedup vs baseline: 1.2038x; 1.2038x over previous
import functools

import jax
import jax.numpy as jnp
from jax import lax
from jax.experimental import pallas as pl
from jax.experimental.pallas import tpu as pltpu
from jax.experimental.pallas import tpu_sc as plsc

NUM_NODES = 4096
H = 16
B = 1024
L = 50
LL = L * L
G = (LL + 15) // 16
GP = 160
LLP = GP * 16
LP = 64
NW = 32
BPW = B // NW
VT = NUM_NODES + 1
HH = H // 2
HALF = HH * LL


def _sc_body(seq_hbm, spd_hbm, embt_hbm, out_hbm,
             seq_v, di_v, mo_v, idx_a, idx_b, spd_a, spd_b, embt_v,
             out_v0, out_v1,
             sem_a, sem_b, sem_o0, sem_o1):
    wid = lax.axis_index("s") * 2 + lax.axis_index("c")
    b0 = wid * BPW
    pltpu.sync_copy(embt_hbm, embt_v)
    pltpu.sync_copy(seq_hbm.at[pl.ds(b0, BPW)], seq_v)
    iota = lax.iota(jnp.int32, 16)

    def qr_body(g, c):
        di, mo = c
        di_v[pl.ds(g * 16, 16)] = di
        mo_v[pl.ds(g * 16, 16)] = mo
        mo2 = mo + 16
        over = mo2 >= L
        return (di + over.astype(jnp.int32), jnp.where(over, mo2 - L, mo2))

    lax.fori_loop(0, GP, qr_body, (iota * 0, iota))

    def build_idx(t, idx_v):
        row = jnp.broadcast_to(t, (16,)).astype(jnp.int32)

        def idx_body(g, c):
            di = di_v[pl.ds(g * 16, 16)]
            mo = mo_v[pl.ds(g * 16, 16)]
            hi = plsc.load_gather(seq_v, [row, di])
            lo = plsc.load_gather(seq_v, [row, mo])
            idx_v[pl.ds(g * 16, 16)] = hi * NUM_NODES + lo
            return c

        lax.fori_loop(0, GP, idx_body, 0)

    def fill_half(spd_v, h0, out_v):
        def g_body(g, c):
            sv = spd_v[pl.ds(g * 16, 16)]
            base = g * 16 + iota
            for hh in range(HH):
                h = h0 + hh
                val = plsc.load_gather(embt_v, [sv + (h * VT)])
                plsc.store_scatter(out_v, [base + (hh * LL)], val)
            return c

        lax.fori_loop(0, G - 1, g_body, 0)

        g = G - 1
        sv = spd_v[pl.ds(g * 16, 16)]
        base = g * 16 + iota
        tmask = iota < (LL - g * 16)
        for hh in range(HH):
            h = h0 + hh
            val = plsc.load_gather(embt_v, [sv + (h * VT)])
            plsc.store_scatter(out_v, [base + (hh * LL)], val, mask=tmask)

    def fire_spd(idx_v, spd_v, sem):
        return pltpu.async_copy(spd_hbm.at[idx_v], spd_v, sem)

    def fire_half(b, half, out_v, sem):
        return pltpu.async_copy(out_v, out_hbm.at[2 * b + half], sem)

    build_idx(0, idx_a)
    fire_spd(idx_a, spd_a, sem_a).wait()

    def pair_body(k, carry):
        te = 2 * k
        to = te + 1
        tn = jnp.minimum(to + 1, BPW - 1)

        build_idx(to, idx_b)
        cp_b = fire_spd(idx_b, spd_b, sem_b)
        fill_half(spd_a, 0, out_v0)
        cp_o0 = fire_half(b0 + te, 0, out_v0, sem_o0)
        fill_half(spd_a, HH, out_v1)
        cp_o1 = fire_half(b0 + te, 1, out_v1, sem_o1)
        cp_b.wait()

        build_idx(tn, idx_a)
        cp_a = fire_spd(idx_a, spd_a, sem_a)
        cp_o0.wait()
        fill_half(spd_b, 0, out_v0)
        cp_o0b = fire_half(b0 + to, 0, out_v0, sem_o0)
        cp_o1.wait()
        fill_half(spd_b, HH, out_v1)
        cp_o1b = fire_half(b0 + to, 1, out_v1, sem_o1)
        cp_a.wait()
        cp_o0b.wait()
        cp_o1b.wait()
        return carry

    lax.fori_loop(0, BPW // 2, pair_body, 0)


@functools.partial(
    pl.kernel,
    mesh=plsc.VectorSubcoreMesh(core_axis_name="c", subcore_axis_name="s"),
    compiler_params=pltpu.CompilerParams(needs_layout_passes=False),
    out_type=jax.ShapeDtypeStruct((2 * B, HALF), jnp.float32),
    scratch_types=[
        pltpu.VMEM((BPW, LP), jnp.int32),
        pltpu.VMEM((LLP,), jnp.int32),
        pltpu.VMEM((LLP,), jnp.int32),
        pltpu.VMEM((LLP,), jnp.int32),
        pltpu.VMEM((LLP,), jnp.int32),
        pltpu.VMEM((LLP,), jnp.int32),
        pltpu.VMEM((LLP,), jnp.int32),
        pltpu.VMEM((H * VT,), jnp.float32),
        pltpu.VMEM((HALF,), jnp.float32),
        pltpu.VMEM((HALF,), jnp.float32),
        pltpu.SemaphoreType.DMA,
        pltpu.SemaphoreType.DMA,
        pltpu.SemaphoreType.DMA,
        pltpu.SemaphoreType.DMA,
    ],
)
def _sc_kernel(*refs):
    _sc_body(*refs)


@jax.jit
def kernel(user_seq, spd_table, emb):
    seq = user_seq.astype(jnp.int32)
    seq_p = jnp.zeros((B, LP), jnp.int32).at[:, :L].set(seq)
    spd_flat = spd_table.reshape(-1)
    embt = emb.T.reshape(-1)
    out = _sc_kernel(seq_p, spd_flat, embt)
    return out.reshape(B, H, L, L)


# speedup vs baseline: 1.7298x; 1.4370x over previous
import functools

import jax
import jax.numpy as jnp
from jax import lax
from jax.experimental import pallas as pl
from jax.experimental.pallas import tpu as pltpu
from jax.experimental.pallas import tpu_sc as plsc

NUM_NODES = 4096
H = 16
B = 1024
L = 50
LL = L * L
G = (LL + 15) // 16
GP = 160
LLP = GP * 16
LP = 64
NW = 32
BPW = B // NW
VT = NUM_NODES + 1


def _sc_body(seq_hbm, spd_hbm, embt_hbm, out_hbm,
             seq_v, di_v, mo_v, idx_a, idx_b, spd_a, spd_b, embt_v, out_v,
             sem_a, sem_b, sem_o):
    wid = lax.axis_index("s") * 2 + lax.axis_index("c")
    b0 = wid * BPW
    pltpu.sync_copy(embt_hbm, embt_v)
    pltpu.sync_copy(seq_hbm.at[pl.ds(b0, BPW)], seq_v)
    iota = lax.iota(jnp.int32, 16)

    def qr_body(g, c):
        di, mo = c
        di_v[pl.ds(g * 16, 16)] = di
        mo_v[pl.ds(g * 16, 16)] = mo
        mo2 = mo + 16
        over = mo2 >= L
        return (di + over.astype(jnp.int32), jnp.where(over, mo2 - L, mo2))

    lax.fori_loop(0, GP, qr_body, (iota * 0, iota))

    def build_idx(t, idx_v):
        row = jnp.broadcast_to(t, (16,)).astype(jnp.int32)

        @plsc.parallel_loop(0, GP, unroll=4)
        def _(g):
            di = di_v[pl.ds(g * 16, 16)]
            mo = mo_v[pl.ds(g * 16, 16)]
            hi = plsc.load_gather(seq_v, [row, di])
            lo = plsc.load_gather(seq_v, [row, mo])
            idx_v[pl.ds(g * 16, 16)] = hi * NUM_NODES + lo

    def fill_batch(spd_v):
        @plsc.parallel_loop(0, G - 1, unroll=2)
        def _(g):
            sv = spd_v[pl.ds(g * 16, 16)]
            base = g * 16 + iota
            for h in range(H):
                val = plsc.load_gather(embt_v, [sv + (h * VT)])
                plsc.store_scatter(out_v, [base + (h * LL)], val)

        g = G - 1
        sv = spd_v[pl.ds(g * 16, 16)]
        base = g * 16 + iota
        tmask = iota < (LL - g * 16)
        for h in range(H):
            val = plsc.load_gather(embt_v, [sv + (h * VT)])
            plsc.store_scatter(out_v, [base + (h * LL)], val, mask=tmask)

    def fire_spd(idx_v, spd_v, sem):
        return pltpu.async_copy(spd_hbm.at[idx_v], spd_v, sem)

    build_idx(0, idx_a)
    fire_spd(idx_a, spd_a, sem_a).wait()

    def pair_body(k, carry):
        te = 2 * k
        to = te + 1
        tn = jnp.minimum(to + 1, BPW - 1)

        build_idx(to, idx_b)
        cp_b = fire_spd(idx_b, spd_b, sem_b)
        fill_batch(spd_a)
        cp_oe = pltpu.async_copy(out_v, out_hbm.at[b0 + te], sem_o)
        cp_b.wait()

        build_idx(tn, idx_a)
        cp_a = fire_spd(idx_a, spd_a, sem_a)
        cp_oe.wait()
        fill_batch(spd_b)
        cp_oo = pltpu.async_copy(out_v, out_hbm.at[b0 + to], sem_o)
        cp_a.wait()
        cp_oo.wait()
        return carry

    lax.fori_loop(0, BPW // 2, pair_body, 0)


@functools.partial(
    pl.kernel,
    mesh=plsc.VectorSubcoreMesh(core_axis_name="c", subcore_axis_name="s"),
    compiler_params=pltpu.CompilerParams(needs_layout_passes=False),
    out_type=jax.ShapeDtypeStruct((B, H * LL), jnp.float32),
    scratch_types=[
        pltpu.VMEM((BPW, LP), jnp.int32),
        pltpu.VMEM((LLP,), jnp.int32),
        pltpu.VMEM((LLP,), jnp.int32),
        pltpu.VMEM((LLP,), jnp.int32),
        pltpu.VMEM((LLP,), jnp.int32),
        pltpu.VMEM((LLP,), jnp.int32),
        pltpu.VMEM((LLP,), jnp.int32),
        pltpu.VMEM((H * VT,), jnp.float32),
        pltpu.VMEM((H * LL,), jnp.float32),
        pltpu.SemaphoreType.DMA,
        pltpu.SemaphoreType.DMA,
        pltpu.SemaphoreType.DMA,
    ],
)
def _sc_kernel(*refs):
    _sc_body(*refs)


@jax.jit
def kernel(user_seq, spd_table, emb):
    seq = user_seq.astype(jnp.int32)
    seq_p = jnp.zeros((B, LP), jnp.int32).at[:, :L].set(seq)
    spd_flat = spd_table.reshape(-1)
    embt = emb.T.reshape(-1)
    out = _sc_kernel(seq_p, spd_flat, embt)
    return out.reshape(B, H, L, L)


# speedup vs baseline: 4.4295x; 2.5606x over previous
import functools

import jax
import jax.numpy as jnp
from jax import lax
from jax.experimental import pallas as pl
from jax.experimental.pallas import tpu as pltpu
from jax.experimental.pallas import tpu_sc as plsc

NUM_NODES = 4096
H = 16
B = 1024
L = 50
LL = L * L
PW = 79
GB = B // 16
NW = 32
VT = NUM_NODES + 1
LPAD = 64


def _sc_body(seqt_hbm, spd_hbm, embt_hbm, out_hbm,
             rsi_a, rsj_a, rsi_b, rsj_b, idx_a, idx_b, spd_a, spd_b,
             embt_v, out_a, out_b,
             sem_sa, sem_sb, sem_r, sem_oa, sem_ob):
    wid = lax.axis_index("s") * 2 + lax.axis_index("c")
    pltpu.sync_copy(embt_hbm, embt_v)

    s0 = wid * PW
    n_w = jnp.minimum(PW, LL - s0)
    i0 = lax.shift_right_logical(s0 * 5243, 18)
    j0 = s0 - i0 * L

    def step(i, j, p):
        adv = (p + 1) < n_w
        jn = j + 1
        wrap = jn >= L
        j2 = jnp.where(wrap, 0, jn)
        i2 = i + wrap.astype(jnp.int32)
        return (jnp.where(adv, i2, i), jnp.where(adv, j2, j),
                jnp.where(adv, p + 1, p))

    def fetch_rows(i, j, rsi, rsj, sem):
        c1 = pltpu.async_copy(seqt_hbm.at[i], rsi, sem)
        c2 = pltpu.async_copy(seqt_hbm.at[j], rsj, sem)
        return c1, c2

    def build_idx(rsi, rsj, idx_v):
        @plsc.parallel_loop(0, GB, unroll=4)
        def _(g):
            si = rsi[pl.ds(g * 16, 16)]
            sj = rsj[pl.ds(g * 16, 16)]
            idx_v[pl.ds(g * 16, 16)] = si * NUM_NODES + sj

    def fill(spd_v, out_v):
        hsplat = [lax.iota(jnp.int32, 16) * 0 + h for h in range(H)]

        @plsc.parallel_loop(0, GB, unroll=4)
        def _(g):
            sv = spd_v[pl.ds(g * 16, 16)]
            base = g * 16 + lax.iota(jnp.int32, 16)
            for h in range(H):
                val = plsc.load_gather(embt_v, [sv + (h * VT)])
                plsc.store_scatter(out_v, [hsplat[h], base], val)

    def fire_spd(idx_v, spd_v, sem):
        return pltpu.async_copy(spd_hbm.at[idx_v], spd_v, sem)

    def fire_out(out_v, i, j, sem):
        return pltpu.async_copy(out_v, out_hbm.at[i, j], sem)

    def drain_out(out_v, sem):
        pltpu.make_async_copy(out_v, out_hbm.at[0, 0], sem).wait()

    c1, c2 = fetch_rows(i0, j0, rsi_a, rsj_a, sem_r)
    c1.wait()
    c2.wait()
    build_idx(rsi_a, rsj_a, idx_a)
    fire_spd(idx_a, spd_a, sem_sa).wait()
    i1, j1, _p1 = step(i0, j0, 0)
    c1, c2 = fetch_rows(i1, j1, rsi_b, rsj_b, sem_r)
    c1.wait()
    c2.wait()
    fire_out(out_a, i0, j0, sem_oa)
    fire_out(out_b, i1, j1, sem_ob)

    def pair_body(m, carry):
        ie, je, pe = carry
        io, jo, po = step(ie, je, pe)
        i2, j2, p2 = step(io, jo, po)
        i3, j3, _p3 = step(i2, j2, p2)

        build_idx(rsi_b, rsj_b, idx_b)
        cp_sb = fire_spd(idx_b, spd_b, sem_sb)
        cr1, cr2 = fetch_rows(i2, j2, rsi_a, rsj_a, sem_r)
        cr3, cr4 = fetch_rows(i3, j3, rsi_b, rsj_b, sem_r)
        drain_out(out_a, sem_oa)
        fill(spd_a, out_a)
        fire_out(out_a, ie, je, sem_oa)
        cp_sb.wait()

        cr1.wait()
        cr2.wait()
        build_idx(rsi_a, rsj_a, idx_a)
        cp_sa = fire_spd(idx_a, spd_a, sem_sa)
        drain_out(out_b, sem_ob)
        fill(spd_b, out_b)
        fire_out(out_b, io, jo, sem_ob)
        cp_sa.wait()
        cr3.wait()
        cr4.wait()
        return (i2, j2, p2)

    lax.fori_loop(0, PW // 2 + 1, pair_body, (i0, j0, jnp.int32(0)))
    drain_out(out_a, sem_oa)
    drain_out(out_b, sem_ob)


@functools.partial(
    pl.kernel,
    mesh=plsc.VectorSubcoreMesh(core_axis_name="c", subcore_axis_name="s"),
    compiler_params=pltpu.CompilerParams(needs_layout_passes=False),
    out_type=jax.ShapeDtypeStruct((L, L, H, B), jnp.float32),
    scratch_types=[
        pltpu.VMEM((B,), jnp.int32),
        pltpu.VMEM((B,), jnp.int32),
        pltpu.VMEM((B,), jnp.int32),
        pltpu.VMEM((B,), jnp.int32),
        pltpu.VMEM((B,), jnp.int32),
        pltpu.VMEM((B,), jnp.int32),
        pltpu.VMEM((B,), jnp.int32),
        pltpu.VMEM((B,), jnp.int32),
        pltpu.VMEM((H * VT,), jnp.float32),
        pltpu.VMEM((H, B), jnp.float32),
        pltpu.VMEM((H, B), jnp.float32),
        pltpu.SemaphoreType.DMA,
        pltpu.SemaphoreType.DMA,
        pltpu.SemaphoreType.DMA,
        pltpu.SemaphoreType.DMA,
        pltpu.SemaphoreType.DMA,
    ],
)
def _sc_kernel(*refs):
    _sc_body(*refs)


@jax.jit
def kernel(user_seq, spd_table, emb):
    seq = user_seq.astype(jnp.int32)
    seqt = jnp.zeros((LPAD, B), jnp.int32).at[:L].set(seq.T)
    spd_flat = spd_table.reshape(-1)
    embt = emb.T.reshape(-1)
    out6 = _sc_kernel(seqt, spd_flat, embt)
    return jnp.transpose(out6, (3, 2, 0, 1))


# speedup vs baseline: 4.6245x; 1.0440x over previous
import functools

import jax
import jax.numpy as jnp
from jax import lax
from jax.experimental import pallas as pl
from jax.experimental.pallas import tpu as pltpu
from jax.experimental.pallas import tpu_sc as plsc

NUM_NODES = 4096
H = 16
B = 1024
L = 50
LL = L * L
PW = 79
GB = B // 16
NW = 32
VT = NUM_NODES + 1
LPAD = 64


def _sc_body(seqt_hbm, spd_hbm, embt_hbm, out_hbm,
             rsi_a, rsj_a, rsi_b, rsj_b, idx_a, idx_b, spd_a, spd_b,
             embt_v, out_a, out_b,
             sem_sa, sem_sb, sem_r, sem_oa, sem_ob):
    wid = lax.axis_index("s") * 2 + lax.axis_index("c")
    pltpu.sync_copy(embt_hbm, embt_v)

    s0 = wid * PW
    n_w = jnp.minimum(PW, LL - s0)
    i0 = lax.shift_right_logical(s0 * 5243, 18)
    j0 = s0 - i0 * L

    def step(i, j, p):
        adv = (p + 1) < n_w
        jn = j + 1
        wrap = jn >= L
        j2 = jnp.where(wrap, 0, jn)
        i2 = i + wrap.astype(jnp.int32)
        return (jnp.where(adv, i2, i), jnp.where(adv, j2, j),
                jnp.where(adv, p + 1, p))

    def fetch_rows(i, j, rsi, rsj, sem):
        c1 = pltpu.async_copy(seqt_hbm.at[i], rsi, sem)
        c2 = pltpu.async_copy(seqt_hbm.at[j], rsj, sem)
        return c1, c2

    def build_idx(rsi, rsj, idx_v):
        @plsc.parallel_loop(0, GB, unroll=4)
        def _(g):
            si = rsi[pl.ds(g * 16, 16)]
            sj = rsj[pl.ds(g * 16, 16)]
            idx_v[pl.ds(g * 16, 16)] = si * NUM_NODES + sj

    def fill(spd_v, out_v):
        hsplat = [lax.iota(jnp.int32, 16) * 0 + h for h in range(H)]

        @plsc.parallel_loop(0, GB, unroll=2)
        def _(g):
            sv = spd_v[pl.ds(g * 16, 16)]
            base = g * 16 + lax.iota(jnp.int32, 16)
            for h in range(H):
                val = plsc.load_gather(embt_v, [sv + (h * VT)])
                plsc.store_scatter(out_v, [hsplat[h], base], val)

    def fire_spd(idx_v, spd_v, sem):
        return pltpu.async_copy(spd_hbm.at[idx_v], spd_v, sem)

    def fire_out(out_v, i, j, sem):
        return pltpu.async_copy(out_v, out_hbm.at[i, j], sem)

    def drain_out(out_v, sem):
        pltpu.make_async_copy(out_v, out_hbm.at[0, 0], sem).wait()

    c1, c2 = fetch_rows(i0, j0, rsi_a, rsj_a, sem_r)
    c1.wait()
    c2.wait()
    build_idx(rsi_a, rsj_a, idx_a)
    fire_spd(idx_a, spd_a, sem_sa).wait()
    i1, j1, _p1 = step(i0, j0, 0)
    c1, c2 = fetch_rows(i1, j1, rsi_b, rsj_b, sem_r)
    c1.wait()
    c2.wait()
    fire_out(out_a, i0, j0, sem_oa)
    fire_out(out_b, i1, j1, sem_ob)

    def pair_body(m, carry):
        ie, je, pe = carry
        io, jo, po = step(ie, je, pe)
        i2, j2, p2 = step(io, jo, po)
        i3, j3, _p3 = step(i2, j2, p2)

        build_idx(rsi_b, rsj_b, idx_b)
        cp_sb = fire_spd(idx_b, spd_b, sem_sb)
        cr1, cr2 = fetch_rows(i2, j2, rsi_a, rsj_a, sem_r)
        cr3, cr4 = fetch_rows(i3, j3, rsi_b, rsj_b, sem_r)
        drain_out(out_a, sem_oa)
        fill(spd_a, out_a)
        fire_out(out_a, ie, je, sem_oa)
        cp_sb.wait()

        cr1.wait()
        cr2.wait()
        build_idx(rsi_a, rsj_a, idx_a)
        cp_sa = fire_spd(idx_a, spd_a, sem_sa)
        drain_out(out_b, sem_ob)
        fill(spd_b, out_b)
        fire_out(out_b, io, jo, sem_ob)
        cp_sa.wait()
        cr3.wait()
        cr4.wait()
        return (i2, j2, p2)

    lax.fori_loop(0, PW // 2 + 1, pair_body, (i0, j0, jnp.int32(0)))
    drain_out(out_a, sem_oa)
    drain_out(out_b, sem_ob)


@functools.partial(
    pl.kernel,
    mesh=plsc.VectorSubcoreMesh(core_axis_name="c", subcore_axis_name="s"),
    compiler_params=pltpu.CompilerParams(needs_layout_passes=False),
    out_type=jax.ShapeDtypeStruct((L, L, H, B), jnp.float32),
    scratch_types=[
        pltpu.VMEM((B,), jnp.int32),
        pltpu.VMEM((B,), jnp.int32),
        pltpu.VMEM((B,), jnp.int32),
        pltpu.VMEM((B,), jnp.int32),
        pltpu.VMEM((B,), jnp.int32),
        pltpu.VMEM((B,), jnp.int32),
        pltpu.VMEM((B,), jnp.int32),
        pltpu.VMEM((B,), jnp.int32),
        pltpu.VMEM((H * VT,), jnp.float32),
        pltpu.VMEM((H, B), jnp.float32),
        pltpu.VMEM((H, B), jnp.float32),
        pltpu.SemaphoreType.DMA,
        pltpu.SemaphoreType.DMA,
        pltpu.SemaphoreType.DMA,
        pltpu.SemaphoreType.DMA,
        pltpu.SemaphoreType.DMA,
    ],
)
def _sc_kernel(*refs):
    _sc_body(*refs)


@jax.jit
def kernel(user_seq, spd_table, emb):
    seq = user_seq.astype(jnp.int32)
    seqt = jnp.zeros((LPAD, B), jnp.int32).at[:L].set(seq.T)
    spd_flat = spd_table.reshape(-1)
    embt = emb.T.reshape(-1)
    out6 = _sc_kernel(seqt, spd_flat, embt)
    return jnp.transpose(out6, (3, 2, 0, 1))
